# Initial kernel scaffold; baseline (speedup 1.0000x reference)
import jax
import jax.numpy as jnp
from jax.experimental import pallas as pl


def kernel(x, edge_index, Wl, bl, Wr, W1, b1, W2, b2, W3, b3):
    raise NotImplementedError("write your pallas kernel here")


# speedup vs baseline: 4.0361x; 4.0361x over previous
import functools

import jax
import jax.numpy as jnp
from jax import lax
from jax.experimental import pallas as pl
from jax.experimental.pallas import tpu as pltpu
from jax.experimental.pallas import tpu_sc as plsc

NC = 2
NS = 16
L = 16
CHUNK = 128
DH = 128


def _round_up(a, b):
    return (a + b - 1) // b * b


def _sc_segment_sum(xcomb, srcoff, dstr, npad, nchunk):
    mesh = plsc.VectorSubcoreMesh(core_axis_name="c", subcore_axis_name="s")
    rows_per_tile = npad // NS
    nzero = rows_per_tile // CHUNK

    @functools.partial(
        pl.kernel,
        out_type=jax.ShapeDtypeStruct((NC, npad, DH), jnp.float32),
        mesh=mesh,
        scratch_types=[
            pltpu.VMEM((nchunk, CHUNK), jnp.int32),
            pltpu.VMEM((nchunk, CHUNK), jnp.int32),
            pltpu.VMEM((CHUNK, DH), jnp.float32),
            pltpu.VMEM_SHARED((npad, DH), jnp.float32),
            pltpu.SemaphoreType.DMA,
        ],
    )
    def k(xcomb_h, srcoff_h, dstr_h, out_h, srcb, dstb, gbuf, acc, sem):
        c = lax.axis_index("c")
        s = lax.axis_index("s")

        def fill_g(i, _):
            for l in range(DH // L):
                gbuf[i, pl.ds(l * L, L)] = jnp.zeros((L,), jnp.float32)
            return 0
        lax.fori_loop(0, CHUNK, fill_g, 0)

        def zero_acc(j, _):
            base = s * rows_per_tile + j * CHUNK
            pltpu.sync_copy(gbuf, acc.at[pl.ds(base, CHUNK)])
            return 0
        lax.fori_loop(0, nzero, zero_acc, 0)

        plsc.subcore_barrier()

        pltpu.sync_copy(srcoff_h.at[c, s], srcb)
        pltpu.sync_copy(dstr_h.at[s], dstb)

        def body(j, _):
            pltpu.async_copy(xcomb_h.at[srcb.at[j]], gbuf, sem).wait()
            pltpu.sync_copy(gbuf, acc.at[dstb.at[j]], add=True)
            return 0
        lax.fori_loop(0, nchunk, body, 0)

        plsc.subcore_barrier()

        base = s * rows_per_tile
        pltpu.sync_copy(acc.at[pl.ds(base, rows_per_tile)],
                        out_h.at[c, pl.ds(base, rows_per_tile)])

    return k(xcomb, srcoff, dstr)


def _sc_counts(dstr2, npad, nchunk2):
    mesh = plsc.VectorSubcoreMesh(core_axis_name="c", subcore_axis_name="s")
    rows_per_tile = npad // NS
    nzero = rows_per_tile // CHUNK

    @functools.partial(
        pl.kernel,
        out_type=jax.ShapeDtypeStruct((NC, npad, DH), jnp.float32),
        mesh=mesh,
        scratch_types=[
            pltpu.VMEM((nchunk2, CHUNK), jnp.int32),
            pltpu.VMEM((CHUNK, DH), jnp.float32),
            pltpu.VMEM_SHARED((npad, DH), jnp.float32),
            pltpu.SemaphoreType.DMA,
        ],
    )
    def k(dstr_h, cnt_h, dstb, onesb, cacc, sem):
        c = lax.axis_index("c")
        s = lax.axis_index("s")

        def fill(i, _):
            for l in range(DH // L):
                onesb[i, pl.ds(l * L, L)] = jnp.zeros((L,), jnp.float32)
            return 0
        lax.fori_loop(0, CHUNK, fill, 0)

        def zero_acc(j, _):
            base = s * rows_per_tile + j * CHUNK
            pltpu.sync_copy(onesb, cacc.at[pl.ds(base, CHUNK)])
            return 0
        lax.fori_loop(0, nzero, zero_acc, 0)

        def fill1(i, _):
            for l in range(DH // L):
                onesb[i, pl.ds(l * L, L)] = jnp.ones((L,), jnp.float32)
            return 0
        lax.fori_loop(0, CHUNK, fill1, 0)

        plsc.subcore_barrier()

        pltpu.sync_copy(dstr_h.at[c, s], dstb)

        def body(j, _):
            pltpu.sync_copy(onesb, cacc.at[dstb.at[j]], add=True)
            return 0
        lax.fori_loop(0, nchunk2, body, 0)

        plsc.subcore_barrier()

        base = s * rows_per_tile
        pltpu.sync_copy(cacc.at[pl.ds(base, rows_per_tile)],
                        cnt_h.at[c, pl.ds(base, rows_per_tile)])

    return k(dstr2)


def _tc_head(s0, s1, cnt, x, wlt0, wlt1, bl, wrt, w1t, b1, w2t, b2, w3t, b3):
    n, d = x.shape
    blk = 2000

    def body(s0_r, s1_r, cnt_r, x_r, wlt0_r, wlt1_r, bl_r, wrt_r,
             w1t_r, b1_r, w2t_r, b2_r, w3t_r, b3_r, out_r):
        rcp = 1.0 / jnp.maximum(cnt_r[...], 1.0)
        m0 = s0_r[...] * rcp
        m1 = s1_r[...] * rcp
        xv = x_r[...]
        conv = (jnp.dot(m0, wlt0_r[...], preferred_element_type=jnp.float32)
                + jnp.dot(m1, wlt1_r[...], preferred_element_type=jnp.float32)
                + bl_r[...]
                + jnp.dot(xv, wrt_r[...], preferred_element_type=jnp.float32))
        h = jnp.maximum(conv, 0.0) + xv
        h = jnp.maximum(jnp.dot(h, w1t_r[...], preferred_element_type=jnp.float32)
                        + b1_r[...], 0.0)
        h = jnp.maximum(jnp.dot(h, w2t_r[...], preferred_element_type=jnp.float32)
                        + b2_r[...], 0.0)
        out_r[...] = jnp.dot(h, w3t_r[...],
                             preferred_element_type=jnp.float32) + b3_r[...]

    full = lambda shape: pl.BlockSpec(shape, lambda i: (0, 0))
    return pl.pallas_call(
        body,
        grid=(n // blk,),
        in_specs=[
            pl.BlockSpec((blk, DH), lambda i: (i, 0)),
            pl.BlockSpec((blk, DH), lambda i: (i, 0)),
            pl.BlockSpec((blk, 1), lambda i: (i, 0)),
            pl.BlockSpec((blk, d), lambda i: (i, 0)),
            full(wlt0.shape), full(wlt1.shape), full(bl.shape), full(wrt.shape),
            full(w1t.shape), full(b1.shape), full(w2t.shape), full(b2.shape),
            full(w3t.shape), full(b3.shape),
        ],
        out_specs=pl.BlockSpec((blk, 1), lambda i: (i, 0)),
        out_shape=jax.ShapeDtypeStruct((n, 1), jnp.float32),
    )(s0, s1, cnt, x, wlt0, wlt1, bl, wrt, w1t, b1, w2t, b2, w3t, b3)


def kernel(x, edge_index, Wl, bl, Wr, W1, b1, W2, b2, W3, b3):
    n, d = x.shape
    e = edge_index.shape[1]
    npad = _round_up(n + 1, NS * CHUNK)
    epad = _round_up(e, NS * CHUNK)
    nchunk = epad // (NS * CHUNK)
    epad2 = _round_up(e, NC * NS * CHUNK)
    nchunk2 = epad2 // (NC * NS * CHUNK)

    src = edge_index[0]
    dst = edge_index[1]

    xh0 = jnp.pad(x[:, :DH], ((0, npad - n), (0, 0)))
    xh1 = jnp.pad(x[:, DH:], ((0, npad - n), (0, 0)))
    xcomb = jnp.concatenate([xh0, xh1], axis=0)

    srcp = jnp.pad(src, (0, epad - e))
    dstp = jnp.pad(dst, (0, epad - e), constant_values=n)
    srcoff = jnp.stack([srcp, srcp + npad]).reshape(NC, NS, nchunk, CHUNK)
    dstr = dstp.reshape(NS, nchunk, CHUNK)

    dstp2 = jnp.pad(dst, (0, epad2 - e), constant_values=n)
    dstr2 = dstp2.reshape(NC, NS, nchunk2, CHUNK)

    summed = _sc_segment_sum(xcomb, srcoff, dstr, npad, nchunk)
    counts = _sc_counts(dstr2, npad, nchunk2)

    cnt = (counts[0, :n, 0] + counts[1, :n, 0]).reshape(n, 1)
    out = _tc_head(
        summed[0, :n], summed[1, :n], cnt, x,
        Wl.T[:DH, :], Wl.T[DH:, :], bl.reshape(1, -1), Wr.T,
        W1.T, b1.reshape(1, -1), W2.T, b2.reshape(1, -1),
        W3.T, b3.reshape(1, -1),
    )
    return out
